# baseline (speedup 1.0000x reference)
import functools

import jax
import jax.numpy as jnp
from jax.experimental import pallas as pl
from jax.experimental.pallas import tpu as pltpu
from jax.experimental.pallas import tpu_sc as plsc

EPS = 1e-12


def _idx_kernel(z_ref, v_in_ref, g_in_ref, b_in_ref, cb_ref, idx_ref,
                *, batch, n_codes):
    f32 = jnp.float32
    v_in = v_in_ref[...]
    w_in = g_in_ref[...] * v_in / jnp.maximum(
        jnp.sqrt(jnp.sum(v_in * v_in, axis=1, keepdims=True)), EPS)
    cb = cb_ref[...]
    cb_n = cb / jnp.maximum(
        jnp.sqrt(jnp.sum(cb * cb, axis=1, keepdims=True)), EPS)
    cb2 = jnp.sum(cb_n * cb_n, axis=1, keepdims=True)

    tt = z_ref.shape[-1]
    iota_k = jax.lax.broadcasted_iota(jnp.int32, (n_codes, tt), 0)

    for b in range(batch):
        zb = z_ref[b]
        ze = jnp.dot(w_in, zb, preferred_element_type=f32) + b_in_ref[...]
        nsq = jnp.sum(ze * ze, axis=0, keepdims=True)
        zen = ze / jnp.maximum(jnp.sqrt(nsq), EPS)
        enc2 = jnp.sum(zen * zen, axis=0, keepdims=True)
        m2 = jnp.dot(cb_n, 2.0 * zen, preferred_element_type=f32)
        scores = -((enc2 - m2) + cb2)
        mx = jnp.max(scores, axis=0, keepdims=True)
        idx_ref[b, :] = jnp.min(
            jnp.where(scores == mx, iota_k, n_codes), axis=0)


def _out_kernel(zq_ref, v_out_ref, g_out_ref, b_out_ref, out_ref, *, batch):
    f32 = jnp.float32
    v_out = v_out_ref[...]
    dc = v_out.shape[1]
    w_out = g_out_ref[...] * v_out / jnp.maximum(
        jnp.sqrt(jnp.sum(v_out * v_out, axis=1, keepdims=True)), EPS)
    for b in range(batch):
        zq_b = zq_ref[b][:, :dc]
        out_ref[b] = jax.lax.dot_general(
            w_out, zq_b, (((1,), (1,)), ((), ())),
            preferred_element_type=f32) + b_out_ref[...]


def _sc_gather(codebook, idx_flat, n, dc):
    mesh = plsc.VectorSubcoreMesh(core_axis_name="c", subcore_axis_name="s")
    nw = 32
    b_per_w = n // nw
    chunk = 128
    n_chunks = b_per_w // chunk
    idx2d = idx_flat.reshape(n // chunk, chunk)
    dcp = 128
    cb_pad = jnp.pad(codebook, ((0, 0), (0, dcp - dc)))

    @functools.partial(
        pl.kernel, mesh=mesh,
        out_type=jax.ShapeDtypeStruct((n, dcp), codebook.dtype),
        scratch_types=[
            pltpu.VMEM((n_chunks, chunk), jnp.int32),
            pltpu.VMEM((b_per_w, dcp), codebook.dtype),
            pltpu.SemaphoreType.DMA,
        ],
    )
    def gather_kernel(cb_hbm, idx_hbm, out_hbm, idx_v, rows_v, sem):
        wid = jax.lax.axis_index("s") * 2 + jax.lax.axis_index("c")
        pltpu.sync_copy(idx_hbm.at[pl.ds(wid * n_chunks, n_chunks)], idx_v)
        copies = []
        for c in range(n_chunks):
            copies.append(pltpu.async_copy(
                cb_hbm.at[idx_v.at[c]],
                rows_v.at[pl.ds(c * chunk, chunk)], sem))
        for cp in copies:
            cp.wait()
        pltpu.sync_copy(rows_v, out_hbm.at[pl.ds(wid * b_per_w, b_per_w)])

    return gather_kernel(cb_pad, idx2d)


def kernel(z, v_in, g_in, b_in, codebook, v_out, g_out, b_out):
    B, Din, T = z.shape
    K, Dc = codebook.shape
    TT = 512
    grid = (T // TT,)
    full = lambda shape: pl.BlockSpec(shape, lambda t: (0,) * len(shape))

    idx = pl.pallas_call(
        functools.partial(_idx_kernel, batch=B, n_codes=K),
        grid=grid,
        in_specs=[
            pl.BlockSpec((B, Din, TT), lambda t: (0, 0, t)),
            full((Dc, Din)),
            full((Dc, 1)),
            full((Dc, 1)),
            full((K, Dc)),
        ],
        out_specs=pl.BlockSpec((B, TT), lambda t: (0, t)),
        out_shape=jax.ShapeDtypeStruct((B, T), jnp.int32),
        compiler_params=pltpu.CompilerParams(
            dimension_semantics=("arbitrary",)),
    )(z, v_in, g_in.reshape(Dc, 1), b_in.reshape(Dc, 1), codebook)

    zq = _sc_gather(codebook, idx.reshape(B * T), B * T, Dc)
    DcP = zq.shape[-1]

    out = pl.pallas_call(
        functools.partial(_out_kernel, batch=B),
        grid=grid,
        in_specs=[
            pl.BlockSpec((B, TT, DcP), lambda t: (0, t, 0)),
            full((Din, Dc)),
            full((Din, 1)),
            full((Din, 1)),
        ],
        out_specs=pl.BlockSpec((B, Din, TT), lambda t: (0, 0, t)),
        out_shape=jax.ShapeDtypeStruct((B, Din, T), jnp.float32),
        compiler_params=pltpu.CompilerParams(
            dimension_semantics=("arbitrary",)),
    )(zq.reshape(B, T, DcP), v_out, g_out.reshape(Din, 1),
      b_out.reshape(Din, 1))
    return out, idx


# speedup vs baseline: 1.0689x; 1.0689x over previous
import functools

import jax
import jax.numpy as jnp
from jax.experimental import pallas as pl
from jax.experimental.pallas import tpu as pltpu
from jax.experimental.pallas import tpu_sc as plsc

EPS = 1e-12


def _idx_kernel(z_ref, v_in_ref, g_in_ref, b_in_ref, cb_ref,
                idx3_ref, *, n_codes):
    f32 = jnp.float32
    v_in = v_in_ref[...]
    w_in = g_in_ref[...] * v_in / jnp.maximum(
        jnp.sqrt(jnp.sum(v_in * v_in, axis=1, keepdims=True)), EPS)
    cb = cb_ref[...]
    cb_n = cb / jnp.maximum(
        jnp.sqrt(jnp.sum(cb * cb, axis=1, keepdims=True)), EPS)
    cb2 = jnp.sum(cb_n * cb_n, axis=1, keepdims=True)

    tt = z_ref.shape[-1]
    iota_k = jax.lax.broadcasted_iota(jnp.int32, (n_codes, tt), 0)

    zb = z_ref[0]
    ze = jnp.dot(w_in, zb, preferred_element_type=f32) + b_in_ref[...]
    nsq = jnp.sum(ze * ze, axis=0, keepdims=True)
    zen = ze / jnp.maximum(jnp.sqrt(nsq), EPS)
    enc2 = jnp.sum(zen * zen, axis=0, keepdims=True)
    m2 = jnp.dot(cb_n, 2.0 * zen, preferred_element_type=f32)
    scores = -((enc2 - m2) + cb2)
    mx = jnp.max(scores, axis=0, keepdims=True)
    idx = jnp.min(jnp.where(scores == mx, iota_k, n_codes), axis=0)
    idx3_ref[0] = idx.reshape(tt // 128, 128)


def _out_kernel(zq_ref, v_out_ref, g_out_ref, b_out_ref, out_ref, *, batch):
    f32 = jnp.float32
    v_out = v_out_ref[...]
    dc = v_out.shape[1]
    w_out = g_out_ref[...] * v_out / jnp.maximum(
        jnp.sqrt(jnp.sum(v_out * v_out, axis=1, keepdims=True)), EPS)
    for b in range(batch):
        zq_b = zq_ref[b][:, :dc]
        out_ref[b] = jax.lax.dot_general(
            w_out, zq_b, (((1,), (1,)), ((), ())),
            preferred_element_type=f32) + b_out_ref[...]


def _sc_gather(codebook, idx_flat, n, dc):
    mesh = plsc.VectorSubcoreMesh(core_axis_name="c", subcore_axis_name="s")
    nw = 32
    b_per_w = n // nw
    chunk = 128
    n_chunks = b_per_w // chunk
    dcp = 128
    cb_pad = jnp.pad(codebook, ((0, 0), (0, dcp - dc)))

    @functools.partial(
        pl.kernel, mesh=mesh,
        out_type=jax.ShapeDtypeStruct((n, dcp), codebook.dtype),
        scratch_types=[
            pltpu.VMEM((b_per_w,), jnp.int32),
            pltpu.VMEM((b_per_w, dcp), codebook.dtype),
            pltpu.SemaphoreType.DMA,
        ],
    )
    def gather_kernel(cb_hbm, idx_hbm, out_hbm, idx_v, rows_v, sem):
        wid = jax.lax.axis_index("s") * 2 + jax.lax.axis_index("c")
        pltpu.sync_copy(idx_hbm.at[pl.ds(wid * b_per_w, b_per_w)], idx_v)
        copies = []
        for c in range(n_chunks):
            copies.append(pltpu.async_copy(
                cb_hbm.at[idx_v.at[pl.ds(c * chunk, chunk)]],
                rows_v.at[pl.ds(c * chunk, chunk)], sem))
        for cp in copies:
            cp.wait()
        pltpu.sync_copy(rows_v, out_hbm.at[pl.ds(wid * b_per_w, b_per_w)])

    return gather_kernel(cb_pad, idx_flat)


def kernel(z, v_in, g_in, b_in, codebook, v_out, g_out, b_out):
    B, Din, T = z.shape
    K, Dc = codebook.shape
    TT = 512
    grid = (T // TT,)
    full = lambda shape: pl.BlockSpec(shape, lambda t: (0,) * len(shape))

    idx3 = pl.pallas_call(
        functools.partial(_idx_kernel, n_codes=K),
        grid=(B,),
        in_specs=[
            pl.BlockSpec((1, Din, T), lambda b: (b, 0, 0)),
            full((Dc, Din)),
            full((Dc, 1)),
            full((Dc, 1)),
            full((K, Dc)),
        ],
        out_specs=pl.BlockSpec((1, T // 128, 128), lambda b: (b, 0, 0)),
        out_shape=jax.ShapeDtypeStruct((B, T // 128, 128), jnp.int32),
        compiler_params=pltpu.CompilerParams(
            dimension_semantics=("arbitrary",)),
    )(z, v_in, g_in.reshape(Dc, 1), b_in.reshape(Dc, 1), codebook)
    idx = idx3.reshape(B, T)

    zq = _sc_gather(codebook, idx3.reshape(B * T), B * T, Dc)
    DcP = zq.shape[-1]

    out = pl.pallas_call(
        functools.partial(_out_kernel, batch=B),
        grid=grid,
        in_specs=[
            pl.BlockSpec((B, TT, DcP), lambda t: (0, t, 0)),
            full((Din, Dc)),
            full((Din, 1)),
            full((Din, 1)),
        ],
        out_specs=pl.BlockSpec((B, Din, TT), lambda t: (0, 0, t)),
        out_shape=jax.ShapeDtypeStruct((B, Din, T), jnp.float32),
        compiler_params=pltpu.CompilerParams(
            dimension_semantics=("arbitrary",)),
    )(zq.reshape(B, T, DcP), v_out, g_out.reshape(Din, 1),
      b_out.reshape(Din, 1))
    return out, idx
